# Initial kernel scaffold; baseline (speedup 1.0000x reference)
import jax
import jax.numpy as jnp
from jax.experimental import pallas as pl


def kernel(boxes):
    raise NotImplementedError("write your pallas kernel here")


# speedup vs baseline: 4.2458x; 4.2458x over previous
import functools

import jax
import jax.numpy as jnp
from jax import lax
from jax.experimental import pallas as pl
from jax.experimental.pallas import tpu as pltpu
from jax.experimental.pallas import tpu_sc as plsc

W = 512
H = 512
B = 20000
NC = 2
NS = 16
NW = NC * NS
ROWS_PER_W = W // NW
BAND = ROWS_PER_W * H
GROUPS = B // 16


def _body(flat_hbm, out_hbm, boxes_v, c0_v, c1_v, c2_v):
    wid = lax.axis_index("s") * NC + lax.axis_index("c")
    row_lo = wid * ROWS_PER_W

    pltpu.sync_copy(flat_hbm, boxes_v)

    def _zero(i, _):
        z = jnp.zeros((16,), jnp.float32)
        c0_v[pl.ds(i * 16, 16)] = z
        c1_v[pl.ds(i * 16, 16)] = z
        c2_v[pl.ds(i * 16, 16)] = z
        return _
    lax.fori_loop(0, BAND // 16, _zero, None)

    lane = lax.iota(jnp.int32, 16)
    ones = jnp.ones((16,), jnp.float32)

    def _scan(g, _):
        pos = g * 64 + lane * 4
        x = plsc.load_gather(boxes_v, [pos])
        y = plsc.load_gather(boxes_v, [pos + 1])
        wv = plsc.load_gather(boxes_v, [pos + 2])
        hv = plsc.load_gather(boxes_v, [pos + 3])
        x0 = (x * jnp.float32(W)).astype(jnp.int32)
        y0 = (y * jnp.float32(H)).astype(jnp.int32)
        m = (x0 >= row_lo) & (x0 < row_lo + ROWS_PER_W)
        li = (x0 - row_lo) * H + y0
        li = jnp.where(m, li, 0)
        plsc.store_scatter(c0_v, [li], ones, mask=m)
        plsc.store_scatter(c1_v, [li], wv, mask=m)
        plsc.store_scatter(c2_v, [li], hv, mask=m)
        return _
    lax.fori_loop(0, GROUPS, _scan, None)

    pltpu.sync_copy(c0_v, out_hbm.at[pl.ds(0 * W * H + wid * BAND, BAND)])
    pltpu.sync_copy(c1_v, out_hbm.at[pl.ds(1 * W * H + wid * BAND, BAND)])
    pltpu.sync_copy(c2_v, out_hbm.at[pl.ds(2 * W * H + wid * BAND, BAND)])


@jax.jit
def _heatmap(flat_boxes):
    mesh = plsc.VectorSubcoreMesh(core_axis_name="c", subcore_axis_name="s")
    run = functools.partial(
        pl.kernel,
        mesh=mesh,
        compiler_params=pltpu.CompilerParams(needs_layout_passes=False),
        out_type=jax.ShapeDtypeStruct((3 * W * H,), jnp.float32),
        scratch_types=[
            pltpu.VMEM((B * 4,), jnp.float32),
            pltpu.VMEM((BAND,), jnp.float32),
            pltpu.VMEM((BAND,), jnp.float32),
            pltpu.VMEM((BAND,), jnp.float32),
        ],
    )(_body)
    return run(flat_boxes)


def kernel(boxes):
    flat = boxes.reshape(-1)
    return _heatmap(flat).reshape(1, 3, W, H)
